# baseline (speedup 1.0000x reference)
import functools

import jax
import jax.numpy as jnp
from jax import lax
from jax.experimental import pallas as pl
from jax.experimental.pallas import tpu as pltpu
from jax.experimental.pallas import tpu_sc as plsc

_VOCAB_SIZE = 1000000
_MOD = _VOCAB_SIZE - 1
_L = 16

_NUM_CORES = 2
_NUM_SUBCORES = 16
_NW = _NUM_CORES * _NUM_SUBCORES

_CHUNK = 128
_PAD = 8


@functools.partial(jax.jit, static_argnames=("seq", "dim"))
def _sc_embed(curr, table, scale16, *, seq, dim):
    n = curr.shape[0]
    per_w = n // _NW
    nchunk = per_w // _CHUNK
    vperchunk = _CHUNK // _L

    mesh = plsc.VectorSubcoreMesh(
        core_axis_name="c", subcore_axis_name="s",
        num_cores=_NUM_CORES, num_subcores=_NUM_SUBCORES)

    @functools.partial(
        pl.kernel,
        out_type=jax.ShapeDtypeStruct((n, dim), jnp.float32),
        mesh=mesh,
        scratch_types=[
            pltpu.VMEM((per_w + 2 * _PAD,), jnp.int32),
            pltpu.VMEM((nchunk, _CHUNK), jnp.int32),
            pltpu.VMEM((7, _CHUNK, dim), jnp.float32),
            pltpu.VMEM((_L,), jnp.float32),
            pltpu.SemaphoreType.DMA((7,)),
            pltpu.SemaphoreType.DMA((7,)),
        ],
    )
    def k(curr_h, table_h, scale_h, out_h,
          tok, idx, rows, sv_ref, gsems, osems):
        wid = lax.axis_index("s") * _NUM_CORES + lax.axis_index("c")
        base = wid * per_w

        @pl.when(wid == 0)
        def _():
            pltpu.sync_copy(curr_h.at[pl.ds(0, per_w + _PAD)],
                            tok.at[pl.ds(_PAD, per_w + _PAD)])

        @pl.when(wid != 0)
        def _():
            pltpu.sync_copy(curr_h.at[pl.ds(base - _PAD, per_w + 2 * _PAD)],
                            tok)

        pltpu.sync_copy(scale_h, sv_ref)
        sv = sv_ref[...]

        nbuf = 7
        depth = 5
        lane = lax.iota(jnp.int32, _L)

        def hash_body(i):
            off = i * _L
            tp = tok[pl.ds(off + _PAD - 1, _L)]
            tc = tok[pl.ds(off + _PAD, _L)]
            h = (jnp.int32(36313) * tc) ^ (jnp.int32(27191) * tp)
            h = h % jnp.int32(_MOD)
            pos = base + off + lane
            h = jnp.where((pos & jnp.int32(seq - 1)) == 0, jnp.int32(_MOD), h)
            c = lax.shift_right_logical(i, 3)
            idx[c, pl.ds((i & (vperchunk - 1)) * _L, _L)] = h

        plsc.parallel_loop(0, vperchunk, unroll=2)(hash_body)
        pltpu.async_copy(table_h.at[idx.at[0]], rows.at[0], gsems.at[0])
        plsc.parallel_loop(vperchunk, per_w // _L, unroll=2)(hash_body)
        for j in range(1, 1 + depth):
            pltpu.async_copy(table_h.at[idx.at[j]], rows.at[j], gsems.at[j])

        def pipe_body(c, _):
            buf = lax.rem(c, nbuf)
            pltpu.make_async_copy(
                table_h.at[idx.at[c]], rows.at[buf], gsems.at[buf]).wait()

            @plsc.parallel_loop(0, _CHUNK, unroll=4)
            def _(r):
                for kk in range(dim // _L):
                    s = pl.ds(kk * _L, _L)
                    rows[buf, r, s] = rows[buf, r, s] * sv

            pltpu.async_copy(
                rows.at[buf],
                out_h.at[pl.ds(base + c * _CHUNK, _CHUNK)],
                osems.at[buf])

            nxt = c + 1 + depth
            @pl.when(nxt < nchunk)
            def _():
                buf2 = lax.rem(nxt, nbuf)

                @pl.when(nxt >= nbuf)
                def _():
                    pltpu.make_async_copy(
                        rows.at[buf2],
                        out_h.at[pl.ds(base + (nxt - nbuf) * _CHUNK, _CHUNK)],
                        osems.at[buf2]).wait()

                pltpu.async_copy(
                    table_h.at[idx.at[nxt]], rows.at[buf2], gsems.at[buf2])

            return 0

        lax.fori_loop(0, nchunk, pipe_body, 0)
        for p in range(nchunk - nbuf, nchunk):
            pb = p % nbuf
            pltpu.make_async_copy(
                rows.at[pb],
                out_h.at[pl.ds(base + p * _CHUNK, _CHUNK)],
                osems.at[pb]).wait()

    return k(curr, table, scale16)


def kernel(token_ids, embed_weight, scale):
    b, seq = token_ids.shape
    dim = embed_weight.shape[1]
    flat = token_ids.reshape(-1).astype(jnp.int32)
    scale16 = jnp.broadcast_to(scale.astype(jnp.float32), (_L,))
    out = _sc_embed(flat, embed_weight, scale16, seq=seq, dim=dim)
    return out.reshape(b, seq, dim)
